# baseline (speedup 1.0000x reference)
import functools

import jax
import jax.numpy as jnp
from jax import lax
from jax.experimental import pallas as pl
from jax.experimental.pallas import tpu as pltpu
from jax.experimental.pallas import tpu_sc as plsc


def kernel(method_soft, label, features):
    del label, features
    n, d = method_soft.shape
    info = plsc.get_sparse_core_info()
    nw = info.num_cores * info.num_subcores
    rows_per = n // nw
    mesh = plsc.VectorSubcoreMesh(core_axis_name="c", subcore_axis_name="s")

    @functools.partial(
        pl.kernel,
        mesh=mesh,
        out_type=jax.ShapeDtypeStruct((n, d), method_soft.dtype),
    )
    def sc_copy(x_hbm, o_hbm):
        wid = lax.axis_index("s") * info.num_cores + lax.axis_index("c")
        base = wid * rows_per
        pltpu.sync_copy(x_hbm.at[pl.ds(base, rows_per)],
                        o_hbm.at[pl.ds(base, rows_per)])

    return sc_copy(method_soft)


# speedup vs baseline: 8.2062x; 8.2062x over previous
import functools

import jax
import jax.numpy as jnp
from jax import lax
from jax.experimental import pallas as pl
from jax.experimental.pallas import tpu as pltpu
from jax.experimental.pallas import tpu_sc as plsc


def kernel(method_soft, label, features):
    del label, features
    n, d = method_soft.shape
    info = plsc.get_sparse_core_info()
    nw = info.num_cores * info.num_subcores
    rows_per = n // nw
    mesh = plsc.VectorSubcoreMesh(core_axis_name="c", subcore_axis_name="s")

    @functools.partial(
        pl.kernel,
        mesh=mesh,
        out_type=jax.ShapeDtypeStruct((n, d), method_soft.dtype),
        scratch_types=[pltpu.VMEM((rows_per, d), method_soft.dtype)],
    )
    def sc_copy(x_hbm, o_hbm, buf):
        wid = lax.axis_index("s") * info.num_cores + lax.axis_index("c")
        base = wid * rows_per
        pltpu.sync_copy(x_hbm.at[pl.ds(base, rows_per)], buf)
        pltpu.sync_copy(buf, o_hbm.at[pl.ds(base, rows_per)])

    return sc_copy(method_soft)


# speedup vs baseline: 15.0683x; 1.8362x over previous
import jax
import jax.numpy as jnp
from jax.experimental import pallas as pl
from jax.experimental.pallas import tpu as pltpu

_K = 8


def _copy_kernel(x_hbm, o_hbm, buf, *sems):
    sems_in, sems_out = sems[:_K], sems[_K:]
    n = x_hbm.shape[0]
    chunk = n // _K
    cps_in = [
        pltpu.make_async_copy(
            x_hbm.at[pl.ds(i * chunk, chunk)],
            buf.at[pl.ds(i * chunk, chunk)],
            sems_in[i],
        )
        for i in range(_K)
    ]
    cps_out = [
        pltpu.make_async_copy(
            buf.at[pl.ds(i * chunk, chunk)],
            o_hbm.at[pl.ds(i * chunk, chunk)],
            sems_out[i],
        )
        for i in range(_K)
    ]
    for cp in cps_in:
        cp.start()
    for i in range(_K):
        cps_in[i].wait()
        cps_out[i].start()
    for cp in cps_out:
        cp.wait()


def kernel(method_soft, label, features):
    del label, features
    return pl.pallas_call(
        _copy_kernel,
        out_shape=jax.ShapeDtypeStruct(method_soft.shape, method_soft.dtype),
        in_specs=[pl.BlockSpec(memory_space=pl.ANY)],
        out_specs=pl.BlockSpec(memory_space=pl.ANY),
        scratch_shapes=[pltpu.VMEM(method_soft.shape, method_soft.dtype)]
        + [pltpu.SemaphoreType.DMA] * (2 * _K),
    )(method_soft)
